# baseline (speedup 1.0000x reference)
import functools

import jax
import jax.numpy as jnp
import numpy as np
from jax.experimental import pallas as pl
from jax.experimental.pallas import tpu as pltpu
from jax.experimental.pallas import tpu_sc as plsc

N_POINTS = 1048576
NUM_LEVELS = 16
BASE_RES = 16
MAX_RES = 2048
HASHMAP = 2 ** 19
HIDDEN = 64
OUT_DIM = 1
PER_LEVEL_SCALE = float(np.exp2(np.log2(MAX_RES / BASE_RES) / (NUM_LEVELS - 1)))
P1 = np.int32(np.uint32(2654435761).astype(np.int64) - (1 << 32))
P2 = np.int32(805459861)

_res, _params, _offs, _scales = [], [], [0], []
for _l in range(NUM_LEVELS):
    _scale = BASE_RES * (PER_LEVEL_SCALE ** _l) - 1.0
    _r = int(np.ceil(_scale)) + 1
    _p = min(HASHMAP, _r ** 3)
    _p = int(np.ceil(_p / 8.0) * 8)
    _res.append(_r)
    _params.append(_p)
    _offs.append(_offs[-1] + _p)
    _scales.append(np.float32(_scale))
TOTAL_PARAMS = _offs[-1]
HASHED = [(_res[l] ** 3) > HASHMAP for l in range(NUM_LEVELS)]

_k = np.arange(128)
_C = _k // 16
_L = _k % 16
CX128 = ((_C >> 2) & 1).astype(np.float32).reshape(1, 128)
CY128 = ((_C >> 1) & 1).astype(np.float32).reshape(1, 128)
CZ128 = (_C & 1).astype(np.float32).reshape(1, 128)
CX128I = CX128.astype(np.int32)
CY128I = CY128.astype(np.int32)
CZ128I = CZ128.astype(np.int32)
SCALE128 = np.asarray([_scales[l] for l in _L], np.float32).reshape(1, 128)
RES128 = np.asarray([_res[l] for l in _L], np.int32).reshape(1, 128)
RES2128 = np.asarray([_res[l] ** 2 for l in _L], np.int32).reshape(1, 128)
RESM1128 = np.asarray([_res[l] - 1 for l in _L], np.int32).reshape(1, 128)
OFF128 = np.asarray([_offs[l] for l in _L], np.int32).reshape(1, 128)
HASH128 = np.asarray([HASHED[l] for l in _L], np.int32).reshape(1, 128)

BLK_A = 1024
BLK_C = 1024
TOTAL_IDX = N_POINTS * NUM_LEVELS * 8
GATHER_WINDOW = 4096
STREAM_CHUNK = 128


def _idx_body(x_ref, scale_ref, cxi_ref, cyi_ref, czi_ref, resm1_ref,
              res_ref, res2_ref, offs_ref, hmask_ref, idx_ref):
    x0 = x_ref[:, 0:1]
    y0 = x_ref[:, 1:2]
    z0 = x_ref[:, 2:3]
    scale = scale_ref[...]
    resm1 = resm1_ref[...]
    ix = jnp.minimum(
        jnp.floor(x0 * scale + 0.5).astype(jnp.int32) + cxi_ref[...], resm1)
    iy = jnp.minimum(
        jnp.floor(y0 * scale + 0.5).astype(jnp.int32) + cyi_ref[...], resm1)
    iz = jnp.minimum(
        jnp.floor(z0 * scale + 0.5).astype(jnp.int32) + czi_ref[...], resm1)
    hidx = (ix ^ (iy * P1) ^ (iz * P2)) & np.int32(HASHMAP - 1)
    didx = ix + iy * res_ref[...] + iz * res2_ref[...]
    idx_ref[...] = jnp.where(hmask_ref[...] != 0, hidx, didx) + offs_ref[...]


def _stage_a(x, n):
    full = lambda i: (0, 0)
    row = pl.BlockSpec((1, 128), full)
    out_blk = pl.BlockSpec((BLK_A, 128), lambda i: (i, 0))
    return pl.pallas_call(
        _idx_body,
        grid=(n // BLK_A,),
        in_specs=[pl.BlockSpec((BLK_A, 3), lambda i: (i, 0)),
                  row, row, row, row, row, row, row, row, row],
        out_specs=out_blk,
        out_shape=jax.ShapeDtypeStruct((n, 128), jnp.int32),
        compiler_params=pltpu.CompilerParams(
            dimension_semantics=("parallel",)),
    )(x, jnp.asarray(SCALE128), jnp.asarray(CX128I), jnp.asarray(CY128I),
      jnp.asarray(CZ128I), jnp.asarray(RESM1128), jnp.asarray(RES128),
      jnp.asarray(RES2128), jnp.asarray(OFF128), jnp.asarray(HASH128))


GATHER_OP = 1024
GATHER_WIN = 8192


def _sc_gather(tab0, tab1, idx_flat, n):
    mesh = plsc.VectorSubcoreMesh(core_axis_name="c", subcore_axis_name="s")
    total = n * 128
    n_ops = GATHER_WIN // GATHER_OP

    @functools.partial(
        pl.kernel,
        out_type=[jax.ShapeDtypeStruct((total,), jnp.float32),
                  jax.ShapeDtypeStruct((total,), jnp.float32)],
        mesh=mesh,
        scratch_types=[pltpu.SemaphoreType.DMA],
    )
    def k(t0_hbm, t1_hbm, i_hbm, o0_hbm, o1_hbm, sem):
        def body(i_vmem, o0_vmem, o1_vmem):
            @pl.loop(0, n_ops)
            def _(j):
                iv = i_vmem.at[0, pl.ds(j * GATHER_OP, GATHER_OP)]
                sl = pl.ds(j * GATHER_OP, GATHER_OP)
                pltpu.async_copy(t0_hbm.at[iv], o0_vmem.at[sl], sem)
                pltpu.async_copy(t1_hbm.at[iv], o1_vmem.at[sl], sem)

            @pl.loop(0, n_ops)
            def _(j):
                iv = i_vmem.at[0, pl.ds(j * GATHER_OP, GATHER_OP)]
                sl = pl.ds(j * GATHER_OP, GATHER_OP)
                pltpu.make_async_copy(t0_hbm.at[iv], o0_vmem.at[sl],
                                      sem).wait()
                pltpu.make_async_copy(t1_hbm.at[iv], o1_vmem.at[sl],
                                      sem).wait()

        pltpu.emit_pipeline(
            body,
            grid=(total // GATHER_WIN,),
            in_specs=[pl.BlockSpec((1, GATHER_WIN),
                                   index_map=lambda i: (0, i))],
            out_specs=[pl.BlockSpec((GATHER_WIN,), index_map=lambda i: (i,)),
                       pl.BlockSpec((GATHER_WIN,), index_map=lambda i: (i,))],
            core_axis_name=("c", "s"),
            dimension_semantics=(pltpu.PARALLEL,),
        )(i_hbm, o0_hbm, o1_hbm)

    return k(tab0, tab1, idx_flat)


def _mlp_body(x_ref, g0_ref, g1_ref, scale_ref, cx_ref, cy_ref, cz_ref,
              m0_ref, m1_ref, b0_ref, w1_ref, b1_ref, w2_ref, b2_ref,
              w3_ref, b3_ref, o_ref):
    x0 = x_ref[:, 0:1]
    y0 = x_ref[:, 1:2]
    z0 = x_ref[:, 2:3]
    scale = scale_ref[...]
    posx = x0 * scale + 0.5
    posy = y0 * scale + 0.5
    posz = z0 * scale + 0.5
    fx = posx - jnp.floor(posx)
    fy = posy - jnp.floor(posy)
    fz = posz - jnp.floor(posz)
    cxm = cx_ref[...]
    cym = cy_ref[...]
    czm = cz_ref[...]
    wx = cxm * fx + (1.0 - cxm) * (1.0 - fx)
    wy = cym * fy + (1.0 - cym) * (1.0 - fy)
    wz = czm * fz + (1.0 - czm) * (1.0 - fz)
    w = wx * wy * wz
    wg0 = w * g0_ref[...]
    wg1 = w * g1_ref[...]
    h = (jnp.dot(wg0, m0_ref[...], preferred_element_type=jnp.float32)
         + jnp.dot(wg1, m1_ref[...], preferred_element_type=jnp.float32)
         + b0_ref[...])
    h = jnp.maximum(h, 0.0)
    h = jnp.maximum(
        jnp.dot(h, w1_ref[...], preferred_element_type=jnp.float32)
        + b1_ref[...], 0.0)
    h = jnp.maximum(
        jnp.dot(h, w2_ref[...], preferred_element_type=jnp.float32)
        + b2_ref[...], 0.0)
    out = jnp.sum(h * w3_ref[...], axis=1, keepdims=True) + b3_ref[...]
    o_ref[...] = out


def _stage_c(x, g0, g1, M0, M1, b0, W1, b1, W2, b2, W3, b3, n):
    full = lambda i: (0, 0)
    return pl.pallas_call(
        _mlp_body,
        grid=(n // BLK_C,),
        in_specs=[
            pl.BlockSpec((BLK_C, 3), lambda i: (i, 0)),
            pl.BlockSpec((BLK_C, 128), lambda i: (i, 0)),
            pl.BlockSpec((BLK_C, 128), lambda i: (i, 0)),
            pl.BlockSpec((1, 128), full),
            pl.BlockSpec((1, 128), full),
            pl.BlockSpec((1, 128), full),
            pl.BlockSpec((1, 128), full),
            pl.BlockSpec((128, HIDDEN), full),
            pl.BlockSpec((128, HIDDEN), full),
            pl.BlockSpec((1, HIDDEN), full),
            pl.BlockSpec((HIDDEN, HIDDEN), full),
            pl.BlockSpec((1, HIDDEN), full),
            pl.BlockSpec((HIDDEN, HIDDEN), full),
            pl.BlockSpec((1, HIDDEN), full),
            pl.BlockSpec((1, HIDDEN), full),
            pl.BlockSpec((1, 1), full),
        ],
        out_specs=pl.BlockSpec((BLK_C, 1), lambda i: (i, 0)),
        out_shape=jax.ShapeDtypeStruct((n, OUT_DIM), jnp.float32),
        compiler_params=pltpu.CompilerParams(
            dimension_semantics=("parallel",)),
    )(x, g0, g1, jnp.asarray(SCALE128), jnp.asarray(CX128),
      jnp.asarray(CY128), jnp.asarray(CZ128), M0, M1, b0, W1, b1, W2, b2,
      W3, b3)


NCHUNK = 4


def kernel(x, table, W0, b0, W1, b1, W2, b2, W3, b3):
    tab0 = table[:, 0]
    tab1 = table[:, 1]
    M0 = jnp.tile(W0[0::2], (8, 1))
    M1 = jnp.tile(W0[1::2], (8, 1))
    nc = N_POINTS // NCHUNK
    outs = []
    for i in range(NCHUNK):
        xs = jax.lax.slice(x, (i * nc, 0), ((i + 1) * nc, 3))
        idx = _stage_a(xs, nc)
        g0, g1 = _sc_gather(tab0, tab1, idx.reshape(1, nc * 128), nc)
        g0 = g0.reshape(nc, 128)
        g1 = g1.reshape(nc, 128)
        outs.append(_stage_c(
            xs, g0, g1, M0, M1, b0.reshape(1, HIDDEN), W1,
            b1.reshape(1, HIDDEN), W2, b2.reshape(1, HIDDEN),
            W3.reshape(1, HIDDEN), b3.reshape(1, 1), nc))
    return jnp.concatenate(outs, axis=0)


# speedup vs baseline: 1.0338x; 1.0338x over previous
import functools

import jax
import jax.numpy as jnp
import numpy as np
from jax.experimental import pallas as pl
from jax.experimental.pallas import tpu as pltpu
from jax.experimental.pallas import tpu_sc as plsc

N_POINTS = 1048576
NUM_LEVELS = 16
BASE_RES = 16
MAX_RES = 2048
HASHMAP = 2 ** 19
HIDDEN = 64
OUT_DIM = 1
PER_LEVEL_SCALE = float(np.exp2(np.log2(MAX_RES / BASE_RES) / (NUM_LEVELS - 1)))
P1 = np.int32(np.uint32(2654435761).astype(np.int64) - (1 << 32))
P2 = np.int32(805459861)

_res, _params, _offs, _scales = [], [], [0], []
for _l in range(NUM_LEVELS):
    _scale = BASE_RES * (PER_LEVEL_SCALE ** _l) - 1.0
    _r = int(np.ceil(_scale)) + 1
    _p = min(HASHMAP, _r ** 3)
    _p = int(np.ceil(_p / 8.0) * 8)
    _res.append(_r)
    _params.append(_p)
    _offs.append(_offs[-1] + _p)
    _scales.append(np.float32(_scale))
TOTAL_PARAMS = _offs[-1]
HASHED = [(_res[l] ** 3) > HASHMAP for l in range(NUM_LEVELS)]

_k = np.arange(128)
_C = _k // 16
_L = _k % 16
CX128 = ((_C >> 2) & 1).astype(np.float32).reshape(1, 128)
CY128 = ((_C >> 1) & 1).astype(np.float32).reshape(1, 128)
CZ128 = (_C & 1).astype(np.float32).reshape(1, 128)
CX128I = CX128.astype(np.int32)
CY128I = CY128.astype(np.int32)
CZ128I = CZ128.astype(np.int32)
SCALE128 = np.asarray([_scales[l] for l in _L], np.float32).reshape(1, 128)
RES128 = np.asarray([_res[l] for l in _L], np.int32).reshape(1, 128)
RES2128 = np.asarray([_res[l] ** 2 for l in _L], np.int32).reshape(1, 128)
RESM1128 = np.asarray([_res[l] - 1 for l in _L], np.int32).reshape(1, 128)
OFF128 = np.asarray([_offs[l] for l in _L], np.int32).reshape(1, 128)
HASH128 = np.asarray([HASHED[l] for l in _L], np.int32).reshape(1, 128)

BLK_A = 1024
BLK_C = 1024
TOTAL_IDX = N_POINTS * NUM_LEVELS * 8
GATHER_WINDOW = 4096
STREAM_CHUNK = 128


def _idx_body(x_ref, scale_ref, cxi_ref, cyi_ref, czi_ref, resm1_ref,
              res_ref, res2_ref, offs_ref, hmask_ref, idx_ref):
    x0 = x_ref[:, 0:1]
    y0 = x_ref[:, 1:2]
    z0 = x_ref[:, 2:3]
    scale = scale_ref[...]
    resm1 = resm1_ref[...]
    ix = jnp.minimum(
        jnp.floor(x0 * scale + 0.5).astype(jnp.int32) + cxi_ref[...], resm1)
    iy = jnp.minimum(
        jnp.floor(y0 * scale + 0.5).astype(jnp.int32) + cyi_ref[...], resm1)
    iz = jnp.minimum(
        jnp.floor(z0 * scale + 0.5).astype(jnp.int32) + czi_ref[...], resm1)
    hidx = (ix ^ (iy * P1) ^ (iz * P2)) & np.int32(HASHMAP - 1)
    didx = ix + iy * res_ref[...] + iz * res2_ref[...]
    idx_ref[...] = jnp.where(hmask_ref[...] != 0, hidx, didx) + offs_ref[...]


def _stage_a(x, n):
    full = lambda i: (0, 0)
    row = pl.BlockSpec((1, 128), full)
    out_blk = pl.BlockSpec((BLK_A, 128), lambda i: (i, 0))
    return pl.pallas_call(
        _idx_body,
        grid=(n // BLK_A,),
        in_specs=[pl.BlockSpec((BLK_A, 3), lambda i: (i, 0)),
                  row, row, row, row, row, row, row, row, row],
        out_specs=out_blk,
        out_shape=jax.ShapeDtypeStruct((n, 128), jnp.int32),
        compiler_params=pltpu.CompilerParams(
            dimension_semantics=("parallel",)),
    )(x, jnp.asarray(SCALE128), jnp.asarray(CX128I), jnp.asarray(CY128I),
      jnp.asarray(CZ128I), jnp.asarray(RESM1128), jnp.asarray(RES128),
      jnp.asarray(RES2128), jnp.asarray(OFF128), jnp.asarray(HASH128))


GATHER_ROWS = 64


def _sc_gather(tab0, tab1, idx, n):
    mesh = plsc.VectorSubcoreMesh(core_axis_name="c", subcore_axis_name="s")

    @functools.partial(
        pl.kernel,
        out_type=[jax.ShapeDtypeStruct((n, 128), jnp.float32),
                  jax.ShapeDtypeStruct((n, 128), jnp.float32)],
        mesh=mesh,
        scratch_types=[pltpu.SemaphoreType.DMA],
    )
    def k(t0_hbm, t1_hbm, i_hbm, o0_hbm, o1_hbm, sem):
        def body(i_vmem, o0_vmem, o1_vmem):
            @pl.loop(0, GATHER_ROWS)
            def _(j):
                pltpu.async_copy(t0_hbm.at[i_vmem.at[j]], o0_vmem.at[j], sem)
                pltpu.async_copy(t1_hbm.at[i_vmem.at[j]], o1_vmem.at[j], sem)

            @pl.loop(0, GATHER_ROWS)
            def _(j):
                pltpu.make_async_copy(
                    t0_hbm.at[i_vmem.at[j]], o0_vmem.at[j], sem).wait()
                pltpu.make_async_copy(
                    t1_hbm.at[i_vmem.at[j]], o1_vmem.at[j], sem).wait()

        blk = pl.BlockSpec((GATHER_ROWS, 128), index_map=lambda i: (i, 0))
        pltpu.emit_pipeline(
            body,
            grid=(n // GATHER_ROWS,),
            in_specs=[blk],
            out_specs=[blk, blk],
            core_axis_name=("c", "s"),
            dimension_semantics=(pltpu.PARALLEL,),
        )(i_hbm, o0_hbm, o1_hbm)

    return k(tab0, tab1, idx)


def _mlp_body(x_ref, g0_ref, g1_ref, scale_ref, cx_ref, cy_ref, cz_ref,
              m0_ref, m1_ref, b0_ref, w1_ref, b1_ref, w2_ref, b2_ref,
              w3_ref, b3_ref, o_ref):
    x0 = x_ref[:, 0:1]
    y0 = x_ref[:, 1:2]
    z0 = x_ref[:, 2:3]
    scale = scale_ref[...]
    posx = x0 * scale + 0.5
    posy = y0 * scale + 0.5
    posz = z0 * scale + 0.5
    fx = posx - jnp.floor(posx)
    fy = posy - jnp.floor(posy)
    fz = posz - jnp.floor(posz)
    cxm = cx_ref[...]
    cym = cy_ref[...]
    czm = cz_ref[...]
    wx = cxm * fx + (1.0 - cxm) * (1.0 - fx)
    wy = cym * fy + (1.0 - cym) * (1.0 - fy)
    wz = czm * fz + (1.0 - czm) * (1.0 - fz)
    w = wx * wy * wz
    wg0 = w * g0_ref[...]
    wg1 = w * g1_ref[...]
    h = (jnp.dot(wg0, m0_ref[...], preferred_element_type=jnp.float32)
         + jnp.dot(wg1, m1_ref[...], preferred_element_type=jnp.float32)
         + b0_ref[...])
    h = jnp.maximum(h, 0.0)
    h = jnp.maximum(
        jnp.dot(h, w1_ref[...], preferred_element_type=jnp.float32)
        + b1_ref[...], 0.0)
    h = jnp.maximum(
        jnp.dot(h, w2_ref[...], preferred_element_type=jnp.float32)
        + b2_ref[...], 0.0)
    out = jnp.sum(h * w3_ref[...], axis=1, keepdims=True) + b3_ref[...]
    o_ref[...] = out


def _stage_c(x, g0, g1, M0, M1, b0, W1, b1, W2, b2, W3, b3, n):
    full = lambda i: (0, 0)
    return pl.pallas_call(
        _mlp_body,
        grid=(n // BLK_C,),
        in_specs=[
            pl.BlockSpec((BLK_C, 3), lambda i: (i, 0)),
            pl.BlockSpec((BLK_C, 128), lambda i: (i, 0)),
            pl.BlockSpec((BLK_C, 128), lambda i: (i, 0)),
            pl.BlockSpec((1, 128), full),
            pl.BlockSpec((1, 128), full),
            pl.BlockSpec((1, 128), full),
            pl.BlockSpec((1, 128), full),
            pl.BlockSpec((128, HIDDEN), full),
            pl.BlockSpec((128, HIDDEN), full),
            pl.BlockSpec((1, HIDDEN), full),
            pl.BlockSpec((HIDDEN, HIDDEN), full),
            pl.BlockSpec((1, HIDDEN), full),
            pl.BlockSpec((HIDDEN, HIDDEN), full),
            pl.BlockSpec((1, HIDDEN), full),
            pl.BlockSpec((1, HIDDEN), full),
            pl.BlockSpec((1, 1), full),
        ],
        out_specs=pl.BlockSpec((BLK_C, 1), lambda i: (i, 0)),
        out_shape=jax.ShapeDtypeStruct((n, OUT_DIM), jnp.float32),
        compiler_params=pltpu.CompilerParams(
            dimension_semantics=("parallel",)),
    )(x, g0, g1, jnp.asarray(SCALE128), jnp.asarray(CX128),
      jnp.asarray(CY128), jnp.asarray(CZ128), M0, M1, b0, W1, b1, W2, b2,
      W3, b3)


NCHUNK = 8


def kernel(x, table, W0, b0, W1, b1, W2, b2, W3, b3):
    tab0 = table[:, 0]
    tab1 = table[:, 1]
    M0 = jnp.tile(W0[0::2], (8, 1))
    M1 = jnp.tile(W0[1::2], (8, 1))
    nc = N_POINTS // NCHUNK
    outs = []
    for i in range(NCHUNK):
        xs = jax.lax.slice(x, (i * nc, 0), ((i + 1) * nc, 3))
        idx = _stage_a(xs, nc)
        g0, g1 = _sc_gather(tab0, tab1, idx, nc)
        outs.append(_stage_c(
            xs, g0, g1, M0, M1, b0.reshape(1, HIDDEN), W1,
            b1.reshape(1, HIDDEN), W2, b2.reshape(1, HIDDEN),
            W3.reshape(1, HIDDEN), b3.reshape(1, 1), nc))
    return jnp.concatenate(outs, axis=0)


# speedup vs baseline: 1.9767x; 1.9120x over previous
import functools

import jax
import jax.numpy as jnp
import numpy as np
from jax.experimental import pallas as pl
from jax.experimental.pallas import tpu as pltpu
from jax.experimental.pallas import tpu_sc as plsc

N_POINTS = 1048576
NUM_LEVELS = 16
BASE_RES = 16
MAX_RES = 2048
HASHMAP = 2 ** 19
HIDDEN = 64
OUT_DIM = 1
PER_LEVEL_SCALE = float(np.exp2(np.log2(MAX_RES / BASE_RES) / (NUM_LEVELS - 1)))
P1 = np.int32(np.uint32(2654435761).astype(np.int64) - (1 << 32))
P2 = np.int32(805459861)

_res, _params, _offs, _scales = [], [], [0], []
for _l in range(NUM_LEVELS):
    _scale = BASE_RES * (PER_LEVEL_SCALE ** _l) - 1.0
    _r = int(np.ceil(_scale)) + 1
    _p = min(HASHMAP, _r ** 3)
    _p = int(np.ceil(_p / 8.0) * 8)
    _res.append(_r)
    _params.append(_p)
    _offs.append(_offs[-1] + _p)
    _scales.append(np.float32(_scale))
TOTAL_PARAMS = _offs[-1]
HASHED = [(_res[l] ** 3) > HASHMAP for l in range(NUM_LEVELS)]

_k = np.arange(128)
_C = _k // 16
_L = _k % 16
CX128 = ((_C >> 2) & 1).astype(np.float32).reshape(1, 128)
CY128 = ((_C >> 1) & 1).astype(np.float32).reshape(1, 128)
CZ128 = (_C & 1).astype(np.float32).reshape(1, 128)
CX128I = CX128.astype(np.int32)
CY128I = CY128.astype(np.int32)
CZ128I = CZ128.astype(np.int32)
SCALE128 = np.asarray([_scales[l] for l in _L], np.float32).reshape(1, 128)
RES128 = np.asarray([_res[l] for l in _L], np.int32).reshape(1, 128)
RES2128 = np.asarray([_res[l] ** 2 for l in _L], np.int32).reshape(1, 128)
RESM1128 = np.asarray([_res[l] - 1 for l in _L], np.int32).reshape(1, 128)
OFF128 = np.asarray([_offs[l] for l in _L], np.int32).reshape(1, 128)
HASH128 = np.asarray([HASHED[l] for l in _L], np.int32).reshape(1, 128)

BLK_A = 1024
BLK_C = 1024
TOTAL_IDX = N_POINTS * NUM_LEVELS * 8
GATHER_WINDOW = 4096
STREAM_CHUNK = 128


def _idx_body(x_ref, scale_ref, cxi_ref, cyi_ref, czi_ref, resm1_ref,
              res_ref, res2_ref, offs_ref, hmask_ref, idx_ref):
    x0 = x_ref[:, 0:1]
    y0 = x_ref[:, 1:2]
    z0 = x_ref[:, 2:3]
    scale = scale_ref[...]
    resm1 = resm1_ref[...]
    ix = jnp.minimum(
        jnp.floor(x0 * scale + 0.5).astype(jnp.int32) + cxi_ref[...], resm1)
    iy = jnp.minimum(
        jnp.floor(y0 * scale + 0.5).astype(jnp.int32) + cyi_ref[...], resm1)
    iz = jnp.minimum(
        jnp.floor(z0 * scale + 0.5).astype(jnp.int32) + czi_ref[...], resm1)
    hidx = (ix ^ (iy * P1) ^ (iz * P2)) & np.int32(HASHMAP - 1)
    didx = ix + iy * res_ref[...] + iz * res2_ref[...]
    idx_ref[...] = jnp.where(hmask_ref[...] != 0, hidx, didx) + offs_ref[...]


def _stage_a(x, n):
    full = lambda i: (0, 0)
    row = pl.BlockSpec((1, 128), full)
    out_blk = pl.BlockSpec((BLK_A, 128), lambda i: (i, 0))
    return pl.pallas_call(
        _idx_body,
        grid=(n // BLK_A,),
        in_specs=[pl.BlockSpec((BLK_A, 3), lambda i: (i, 0)),
                  row, row, row, row, row, row, row, row, row],
        out_specs=out_blk,
        out_shape=jax.ShapeDtypeStruct((n, 128), jnp.int32),
        compiler_params=pltpu.CompilerParams(
            dimension_semantics=("parallel",)),
    )(x, jnp.asarray(SCALE128), jnp.asarray(CX128I), jnp.asarray(CY128I),
      jnp.asarray(CZ128I), jnp.asarray(RESM1128), jnp.asarray(RES128),
      jnp.asarray(RES2128), jnp.asarray(OFF128), jnp.asarray(HASH128))


GATHER_ROWS = 32


def _sc_gather(tab0, tab1, idx, n):
    mesh = plsc.VectorSubcoreMesh(core_axis_name="c", subcore_axis_name="s")

    @functools.partial(
        pl.kernel,
        out_type=[jax.ShapeDtypeStruct((n, 128), jnp.float32),
                  jax.ShapeDtypeStruct((n, 128), jnp.float32)],
        mesh=mesh,
        scratch_types=[pltpu.SemaphoreType.DMA],
    )
    def k(t0_hbm, t1_hbm, i_hbm, o0_hbm, o1_hbm, sem):
        def body(i_vmem, o0_vmem, o1_vmem):
            @pl.loop(0, GATHER_ROWS)
            def _(j):
                pltpu.async_copy(t0_hbm.at[i_vmem.at[j]], o0_vmem.at[j], sem)
                pltpu.async_copy(t1_hbm.at[i_vmem.at[j]], o1_vmem.at[j], sem)

            @pl.loop(0, GATHER_ROWS)
            def _(j):
                pltpu.make_async_copy(
                    t0_hbm.at[i_vmem.at[j]], o0_vmem.at[j], sem).wait()
                pltpu.make_async_copy(
                    t1_hbm.at[i_vmem.at[j]], o1_vmem.at[j], sem).wait()

        blk = pl.BlockSpec((GATHER_ROWS, 128), index_map=lambda i: (i, 0))
        pltpu.emit_pipeline(
            body,
            grid=(n // GATHER_ROWS,),
            in_specs=[blk],
            out_specs=[blk, blk],
            core_axis_name=("c", "s"),
            dimension_semantics=(pltpu.PARALLEL,),
        )(i_hbm, o0_hbm, o1_hbm)

    return k(tab0, tab1, idx)


def _mlp_body(x_ref, g0_ref, g1_ref, scale_ref, cx_ref, cy_ref, cz_ref,
              m0_ref, m1_ref, b0_ref, w1_ref, b1_ref, w2_ref, b2_ref,
              w3_ref, b3_ref, o_ref):
    x0 = x_ref[:, 0:1]
    y0 = x_ref[:, 1:2]
    z0 = x_ref[:, 2:3]
    scale = scale_ref[...]
    posx = x0 * scale + 0.5
    posy = y0 * scale + 0.5
    posz = z0 * scale + 0.5
    fx = posx - jnp.floor(posx)
    fy = posy - jnp.floor(posy)
    fz = posz - jnp.floor(posz)
    cxm = cx_ref[...]
    cym = cy_ref[...]
    czm = cz_ref[...]
    wx = cxm * fx + (1.0 - cxm) * (1.0 - fx)
    wy = cym * fy + (1.0 - cym) * (1.0 - fy)
    wz = czm * fz + (1.0 - czm) * (1.0 - fz)
    w = wx * wy * wz
    wg0 = w * g0_ref[...]
    wg1 = w * g1_ref[...]
    h = (jnp.dot(wg0, m0_ref[...], preferred_element_type=jnp.float32)
         + jnp.dot(wg1, m1_ref[...], preferred_element_type=jnp.float32)
         + b0_ref[...])
    h = jnp.maximum(h, 0.0)
    h = jnp.maximum(
        jnp.dot(h, w1_ref[...], preferred_element_type=jnp.float32)
        + b1_ref[...], 0.0)
    h = jnp.maximum(
        jnp.dot(h, w2_ref[...], preferred_element_type=jnp.float32)
        + b2_ref[...], 0.0)
    out = jnp.sum(h * w3_ref[...], axis=1, keepdims=True) + b3_ref[...]
    o_ref[...] = out


def _stage_c(x, g0, g1, M0, M1, b0, W1, b1, W2, b2, W3, b3, n):
    full = lambda i: (0, 0)
    return pl.pallas_call(
        _mlp_body,
        grid=(n // BLK_C,),
        in_specs=[
            pl.BlockSpec((BLK_C, 3), lambda i: (i, 0)),
            pl.BlockSpec((BLK_C, 128), lambda i: (i, 0)),
            pl.BlockSpec((BLK_C, 128), lambda i: (i, 0)),
            pl.BlockSpec((1, 128), full),
            pl.BlockSpec((1, 128), full),
            pl.BlockSpec((1, 128), full),
            pl.BlockSpec((1, 128), full),
            pl.BlockSpec((128, HIDDEN), full),
            pl.BlockSpec((128, HIDDEN), full),
            pl.BlockSpec((1, HIDDEN), full),
            pl.BlockSpec((HIDDEN, HIDDEN), full),
            pl.BlockSpec((1, HIDDEN), full),
            pl.BlockSpec((HIDDEN, HIDDEN), full),
            pl.BlockSpec((1, HIDDEN), full),
            pl.BlockSpec((1, HIDDEN), full),
            pl.BlockSpec((1, 1), full),
        ],
        out_specs=pl.BlockSpec((BLK_C, 1), lambda i: (i, 0)),
        out_shape=jax.ShapeDtypeStruct((n, OUT_DIM), jnp.float32),
        compiler_params=pltpu.CompilerParams(
            dimension_semantics=("parallel",)),
    )(x, g0, g1, jnp.asarray(SCALE128), jnp.asarray(CX128),
      jnp.asarray(CY128), jnp.asarray(CZ128), M0, M1, b0, W1, b1, W2, b2,
      W3, b3)


NCHUNK = 4


def kernel(x, table, W0, b0, W1, b1, W2, b2, W3, b3):
    tab0 = table[:, 0]
    tab1 = table[:, 1]
    M0 = jnp.tile(W0[0::2], (8, 1))
    M1 = jnp.tile(W0[1::2], (8, 1))
    nc = N_POINTS // NCHUNK
    outs = []
    for i in range(NCHUNK):
        xs = jax.lax.slice(x, (i * nc, 0), ((i + 1) * nc, 3))
        idx = _stage_a(xs, nc)
        g0, g1 = _sc_gather(tab0, tab1, idx, nc)
        outs.append(_stage_c(
            xs, g0, g1, M0, M1, b0.reshape(1, HIDDEN), W1,
            b1.reshape(1, HIDDEN), W2, b2.reshape(1, HIDDEN),
            W3.reshape(1, HIDDEN), b3.reshape(1, 1), nc))
    return jnp.concatenate(outs, axis=0)


# speedup vs baseline: 3.7822x; 1.9134x over previous
import functools

import jax
import jax.numpy as jnp
import numpy as np
from jax.experimental import pallas as pl
from jax.experimental.pallas import tpu as pltpu
from jax.experimental.pallas import tpu_sc as plsc

N_POINTS = 1048576
NUM_LEVELS = 16
BASE_RES = 16
MAX_RES = 2048
HASHMAP = 2 ** 19
HIDDEN = 64
OUT_DIM = 1
PER_LEVEL_SCALE = float(np.exp2(np.log2(MAX_RES / BASE_RES) / (NUM_LEVELS - 1)))
P1 = np.int32(np.uint32(2654435761).astype(np.int64) - (1 << 32))
P2 = np.int32(805459861)

_res, _params, _offs, _scales = [], [], [0], []
for _l in range(NUM_LEVELS):
    _scale = BASE_RES * (PER_LEVEL_SCALE ** _l) - 1.0
    _r = int(np.ceil(_scale)) + 1
    _p = min(HASHMAP, _r ** 3)
    _p = int(np.ceil(_p / 8.0) * 8)
    _res.append(_r)
    _params.append(_p)
    _offs.append(_offs[-1] + _p)
    _scales.append(np.float32(_scale))
TOTAL_PARAMS = _offs[-1]
HASHED = [(_res[l] ** 3) > HASHMAP for l in range(NUM_LEVELS)]

_k = np.arange(128)
_C = _k // 16
_L = _k % 16
CX128 = ((_C >> 2) & 1).astype(np.float32).reshape(1, 128)
CY128 = ((_C >> 1) & 1).astype(np.float32).reshape(1, 128)
CZ128 = (_C & 1).astype(np.float32).reshape(1, 128)
CX128I = CX128.astype(np.int32)
CY128I = CY128.astype(np.int32)
CZ128I = CZ128.astype(np.int32)
SCALE128 = np.asarray([_scales[l] for l in _L], np.float32).reshape(1, 128)
RES128 = np.asarray([_res[l] for l in _L], np.int32).reshape(1, 128)
RES2128 = np.asarray([_res[l] ** 2 for l in _L], np.int32).reshape(1, 128)
RESM1128 = np.asarray([_res[l] - 1 for l in _L], np.int32).reshape(1, 128)
OFF128 = np.asarray([_offs[l] for l in _L], np.int32).reshape(1, 128)
HASH128 = np.asarray([HASHED[l] for l in _L], np.int32).reshape(1, 128)

BLK_A = 1024
BLK_C = 1024


def _idx_body(x_ref, scale_ref, cxi_ref, cyi_ref, czi_ref, resm1_ref,
              res_ref, res2_ref, offs_ref, hmask_ref, idx_ref):
    x0 = x_ref[:, 0:1]
    y0 = x_ref[:, 1:2]
    z0 = x_ref[:, 2:3]
    scale = scale_ref[...]
    resm1 = resm1_ref[...]
    ix = jnp.minimum(
        jnp.floor(x0 * scale + 0.5).astype(jnp.int32) + cxi_ref[...], resm1)
    iy = jnp.minimum(
        jnp.floor(y0 * scale + 0.5).astype(jnp.int32) + cyi_ref[...], resm1)
    iz = jnp.minimum(
        jnp.floor(z0 * scale + 0.5).astype(jnp.int32) + czi_ref[...], resm1)
    hidx = (ix ^ (iy * P1) ^ (iz * P2)) & np.int32(HASHMAP - 1)
    didx = ix + iy * res_ref[...] + iz * res2_ref[...]
    idx_ref[...] = jnp.where(hmask_ref[...] != 0, hidx, didx) + offs_ref[...]


def _stage_a(x, n):
    full = lambda i: (0, 0)
    row = pl.BlockSpec((1, 128), full)
    out_blk = pl.BlockSpec((BLK_A, 128), lambda i: (i, 0))
    return pl.pallas_call(
        _idx_body,
        grid=(n // BLK_A,),
        in_specs=[pl.BlockSpec((BLK_A, 3), lambda i: (i, 0)),
                  row, row, row, row, row, row, row, row, row],
        out_specs=out_blk,
        out_shape=jax.ShapeDtypeStruct((n, 128), jnp.int32),
        compiler_params=pltpu.CompilerParams(
            dimension_semantics=("parallel",)),
    )(x, jnp.asarray(SCALE128), jnp.asarray(CX128I), jnp.asarray(CY128I),
      jnp.asarray(CZ128I), jnp.asarray(RESM1128), jnp.asarray(RES128),
      jnp.asarray(RES2128), jnp.asarray(OFF128), jnp.asarray(HASH128))


GATHER_ROWS = 16


def _sc_gather(tab0, tab1, idx, n):
    mesh = plsc.VectorSubcoreMesh(core_axis_name="c", subcore_axis_name="s")

    @functools.partial(
        pl.kernel,
        out_type=[jax.ShapeDtypeStruct((n, 128), jnp.float32),
                  jax.ShapeDtypeStruct((n, 128), jnp.float32)],
        mesh=mesh,
        scratch_types=[pltpu.SemaphoreType.DMA],
    )
    def k(t0_hbm, t1_hbm, i_hbm, o0_hbm, o1_hbm, sem):
        def body(i_vmem, o0_vmem, o1_vmem):
            @pl.loop(0, GATHER_ROWS)
            def _(j):
                pltpu.async_copy(t0_hbm.at[i_vmem.at[j]], o0_vmem.at[j], sem)
                pltpu.async_copy(t1_hbm.at[i_vmem.at[j]], o1_vmem.at[j], sem)

            @pl.loop(0, GATHER_ROWS)
            def _(j):
                pltpu.make_async_copy(
                    t0_hbm.at[i_vmem.at[j]], o0_vmem.at[j], sem).wait()
                pltpu.make_async_copy(
                    t1_hbm.at[i_vmem.at[j]], o1_vmem.at[j], sem).wait()

        blk = pl.BlockSpec((GATHER_ROWS, 128), index_map=lambda i: (i, 0))
        pltpu.emit_pipeline(
            body,
            grid=(n // GATHER_ROWS,),
            in_specs=[blk],
            out_specs=[blk, blk],
            core_axis_name=("c", "s"),
            dimension_semantics=(pltpu.PARALLEL,),
        )(i_hbm, o0_hbm, o1_hbm)

    return k(tab0, tab1, idx)


def _mlp_body(x_ref, g0_ref, g1_ref, scale_ref, cx_ref, cy_ref, cz_ref,
              m0_ref, m1_ref, b0_ref, w1_ref, b1_ref, w2_ref, b2_ref,
              w3_ref, b3_ref, o_ref):
    x0 = x_ref[:, 0:1]
    y0 = x_ref[:, 1:2]
    z0 = x_ref[:, 2:3]
    scale = scale_ref[...]
    posx = x0 * scale + 0.5
    posy = y0 * scale + 0.5
    posz = z0 * scale + 0.5
    fx = posx - jnp.floor(posx)
    fy = posy - jnp.floor(posy)
    fz = posz - jnp.floor(posz)
    cxm = cx_ref[...]
    cym = cy_ref[...]
    czm = cz_ref[...]
    wx = cxm * fx + (1.0 - cxm) * (1.0 - fx)
    wy = cym * fy + (1.0 - cym) * (1.0 - fy)
    wz = czm * fz + (1.0 - czm) * (1.0 - fz)
    w = wx * wy * wz
    wg0 = w * g0_ref[...]
    wg1 = w * g1_ref[...]
    h = (jnp.dot(wg0, m0_ref[...], preferred_element_type=jnp.float32)
         + jnp.dot(wg1, m1_ref[...], preferred_element_type=jnp.float32)
         + b0_ref[...])
    h = jnp.maximum(h, 0.0)
    h = jnp.maximum(
        jnp.dot(h, w1_ref[...], preferred_element_type=jnp.float32)
        + b1_ref[...], 0.0)
    h = jnp.maximum(
        jnp.dot(h, w2_ref[...], preferred_element_type=jnp.float32)
        + b2_ref[...], 0.0)
    out = jnp.sum(h * w3_ref[...], axis=1, keepdims=True) + b3_ref[...]
    o_ref[...] = out


def _stage_c(x, g0, g1, M0, M1, b0, W1, b1, W2, b2, W3, b3, n):
    full = lambda i: (0, 0)
    return pl.pallas_call(
        _mlp_body,
        grid=(n // BLK_C,),
        in_specs=[
            pl.BlockSpec((BLK_C, 3), lambda i: (i, 0)),
            pl.BlockSpec((BLK_C, 128), lambda i: (i, 0)),
            pl.BlockSpec((BLK_C, 128), lambda i: (i, 0)),
            pl.BlockSpec((1, 128), full),
            pl.BlockSpec((1, 128), full),
            pl.BlockSpec((1, 128), full),
            pl.BlockSpec((1, 128), full),
            pl.BlockSpec((128, HIDDEN), full),
            pl.BlockSpec((128, HIDDEN), full),
            pl.BlockSpec((1, HIDDEN), full),
            pl.BlockSpec((HIDDEN, HIDDEN), full),
            pl.BlockSpec((1, HIDDEN), full),
            pl.BlockSpec((HIDDEN, HIDDEN), full),
            pl.BlockSpec((1, HIDDEN), full),
            pl.BlockSpec((1, HIDDEN), full),
            pl.BlockSpec((1, 1), full),
        ],
        out_specs=pl.BlockSpec((BLK_C, 1), lambda i: (i, 0)),
        out_shape=jax.ShapeDtypeStruct((n, OUT_DIM), jnp.float32),
        compiler_params=pltpu.CompilerParams(
            dimension_semantics=("parallel",)),
    )(x, g0, g1, jnp.asarray(SCALE128), jnp.asarray(CX128),
      jnp.asarray(CY128), jnp.asarray(CZ128), M0, M1, b0, W1, b1, W2, b2,
      W3, b3)


NCHUNK = 4


def kernel(x, table, W0, b0, W1, b1, W2, b2, W3, b3):
    tab0 = table[:, 0]
    tab1 = table[:, 1]
    M0 = jnp.tile(W0[0::2], (8, 1))
    M1 = jnp.tile(W0[1::2], (8, 1))
    nc = N_POINTS // NCHUNK
    outs = []
    for i in range(NCHUNK):
        xs = jax.lax.slice(x, (i * nc, 0), ((i + 1) * nc, 3))
        idx = _stage_a(xs, nc)
        g0, g1 = _sc_gather(tab0, tab1, idx, nc)
        outs.append(_stage_c(
            xs, g0, g1, M0, M1, b0.reshape(1, HIDDEN), W1,
            b1.reshape(1, HIDDEN), W2, b2.reshape(1, HIDDEN),
            W3.reshape(1, HIDDEN), b3.reshape(1, 1), nc))
    return jnp.concatenate(outs, axis=0)


# speedup vs baseline: 6.7797x; 1.7925x over previous
import functools

import jax
import jax.numpy as jnp
import numpy as np
from jax.experimental import pallas as pl
from jax.experimental.pallas import tpu as pltpu
from jax.experimental.pallas import tpu_sc as plsc

N_POINTS = 1048576
NUM_LEVELS = 16
BASE_RES = 16
MAX_RES = 2048
HASHMAP = 2 ** 19
HIDDEN = 64
OUT_DIM = 1
PER_LEVEL_SCALE = float(np.exp2(np.log2(MAX_RES / BASE_RES) / (NUM_LEVELS - 1)))
P1 = np.int32(np.uint32(2654435761).astype(np.int64) - (1 << 32))
P2 = np.int32(805459861)

_res, _params, _offs, _scales = [], [], [0], []
for _l in range(NUM_LEVELS):
    _scale = BASE_RES * (PER_LEVEL_SCALE ** _l) - 1.0
    _r = int(np.ceil(_scale)) + 1
    _p = min(HASHMAP, _r ** 3)
    _p = int(np.ceil(_p / 8.0) * 8)
    _res.append(_r)
    _params.append(_p)
    _offs.append(_offs[-1] + _p)
    _scales.append(np.float32(_scale))
TOTAL_PARAMS = _offs[-1]
HASHED = [(_res[l] ** 3) > HASHMAP for l in range(NUM_LEVELS)]

_k = np.arange(128)
_C = _k // 16
_L = _k % 16
CX128 = ((_C >> 2) & 1).astype(np.float32).reshape(1, 128)
CY128 = ((_C >> 1) & 1).astype(np.float32).reshape(1, 128)
CZ128 = (_C & 1).astype(np.float32).reshape(1, 128)
CX128I = CX128.astype(np.int32)
CY128I = CY128.astype(np.int32)
CZ128I = CZ128.astype(np.int32)
SCALE128 = np.asarray([_scales[l] for l in _L], np.float32).reshape(1, 128)
RES128 = np.asarray([_res[l] for l in _L], np.int32).reshape(1, 128)
RES2128 = np.asarray([_res[l] ** 2 for l in _L], np.int32).reshape(1, 128)
RESM1128 = np.asarray([_res[l] - 1 for l in _L], np.int32).reshape(1, 128)
OFF128 = np.asarray([_offs[l] for l in _L], np.int32).reshape(1, 128)
HASH128 = np.asarray([HASHED[l] for l in _L], np.int32).reshape(1, 128)

BLK_A = 1024
BLK_C = 1024


def _idx_body(x_ref, scale_ref, cxi_ref, cyi_ref, czi_ref, resm1_ref,
              res_ref, res2_ref, offs_ref, hmask_ref, idx_ref):
    x0 = x_ref[:, 0:1]
    y0 = x_ref[:, 1:2]
    z0 = x_ref[:, 2:3]
    scale = scale_ref[...]
    resm1 = resm1_ref[...]
    ix = jnp.minimum(
        jnp.floor(x0 * scale + 0.5).astype(jnp.int32) + cxi_ref[...], resm1)
    iy = jnp.minimum(
        jnp.floor(y0 * scale + 0.5).astype(jnp.int32) + cyi_ref[...], resm1)
    iz = jnp.minimum(
        jnp.floor(z0 * scale + 0.5).astype(jnp.int32) + czi_ref[...], resm1)
    hidx = (ix ^ (iy * P1) ^ (iz * P2)) & np.int32(HASHMAP - 1)
    didx = ix + iy * res_ref[...] + iz * res2_ref[...]
    idx_ref[...] = jnp.where(hmask_ref[...] != 0, hidx, didx) + offs_ref[...]


def _stage_a(x, n):
    full = lambda i: (0, 0)
    row = pl.BlockSpec((1, 128), full)
    out_blk = pl.BlockSpec((BLK_A, 128), lambda i: (i, 0))
    return pl.pallas_call(
        _idx_body,
        grid=(n // BLK_A,),
        in_specs=[pl.BlockSpec((BLK_A, 3), lambda i: (i, 0)),
                  row, row, row, row, row, row, row, row, row],
        out_specs=out_blk,
        out_shape=jax.ShapeDtypeStruct((n, 128), jnp.int32),
        compiler_params=pltpu.CompilerParams(
            dimension_semantics=("parallel",)),
    )(x, jnp.asarray(SCALE128), jnp.asarray(CX128I), jnp.asarray(CY128I),
      jnp.asarray(CZ128I), jnp.asarray(RESM1128), jnp.asarray(RES128),
      jnp.asarray(RES2128), jnp.asarray(OFF128), jnp.asarray(HASH128))


GATHER_ROWS = 8


def _sc_gather(tab0, tab1, idx, n):
    mesh = plsc.VectorSubcoreMesh(core_axis_name="c", subcore_axis_name="s")

    @functools.partial(
        pl.kernel,
        out_type=[jax.ShapeDtypeStruct((n, 128), jnp.float32),
                  jax.ShapeDtypeStruct((n, 128), jnp.float32)],
        mesh=mesh,
        scratch_types=[pltpu.SemaphoreType.DMA],
    )
    def k(t0_hbm, t1_hbm, i_hbm, o0_hbm, o1_hbm, sem):
        def body(i_vmem, o0_vmem, o1_vmem):
            @pl.loop(0, GATHER_ROWS)
            def _(j):
                pltpu.async_copy(t0_hbm.at[i_vmem.at[j]], o0_vmem.at[j], sem)
                pltpu.async_copy(t1_hbm.at[i_vmem.at[j]], o1_vmem.at[j], sem)

            @pl.loop(0, GATHER_ROWS)
            def _(j):
                pltpu.make_async_copy(
                    t0_hbm.at[i_vmem.at[j]], o0_vmem.at[j], sem).wait()
                pltpu.make_async_copy(
                    t1_hbm.at[i_vmem.at[j]], o1_vmem.at[j], sem).wait()

        blk = pl.BlockSpec((GATHER_ROWS, 128), index_map=lambda i: (i, 0))
        pltpu.emit_pipeline(
            body,
            grid=(n // GATHER_ROWS,),
            in_specs=[blk],
            out_specs=[blk, blk],
            core_axis_name=("c", "s"),
            dimension_semantics=(pltpu.PARALLEL,),
        )(i_hbm, o0_hbm, o1_hbm)

    return k(tab0, tab1, idx)


def _mlp_body(x_ref, g0_ref, g1_ref, scale_ref, cx_ref, cy_ref, cz_ref,
              m0_ref, m1_ref, b0_ref, w1_ref, b1_ref, w2_ref, b2_ref,
              w3_ref, b3_ref, o_ref):
    x0 = x_ref[:, 0:1]
    y0 = x_ref[:, 1:2]
    z0 = x_ref[:, 2:3]
    scale = scale_ref[...]
    posx = x0 * scale + 0.5
    posy = y0 * scale + 0.5
    posz = z0 * scale + 0.5
    fx = posx - jnp.floor(posx)
    fy = posy - jnp.floor(posy)
    fz = posz - jnp.floor(posz)
    cxm = cx_ref[...]
    cym = cy_ref[...]
    czm = cz_ref[...]
    wx = cxm * fx + (1.0 - cxm) * (1.0 - fx)
    wy = cym * fy + (1.0 - cym) * (1.0 - fy)
    wz = czm * fz + (1.0 - czm) * (1.0 - fz)
    w = wx * wy * wz
    wg0 = w * g0_ref[...]
    wg1 = w * g1_ref[...]
    h = (jnp.dot(wg0, m0_ref[...], preferred_element_type=jnp.float32)
         + jnp.dot(wg1, m1_ref[...], preferred_element_type=jnp.float32)
         + b0_ref[...])
    h = jnp.maximum(h, 0.0)
    h = jnp.maximum(
        jnp.dot(h, w1_ref[...], preferred_element_type=jnp.float32)
        + b1_ref[...], 0.0)
    h = jnp.maximum(
        jnp.dot(h, w2_ref[...], preferred_element_type=jnp.float32)
        + b2_ref[...], 0.0)
    out = jnp.sum(h * w3_ref[...], axis=1, keepdims=True) + b3_ref[...]
    o_ref[...] = out


def _stage_c(x, g0, g1, M0, M1, b0, W1, b1, W2, b2, W3, b3, n):
    full = lambda i: (0, 0)
    return pl.pallas_call(
        _mlp_body,
        grid=(n // BLK_C,),
        in_specs=[
            pl.BlockSpec((BLK_C, 3), lambda i: (i, 0)),
            pl.BlockSpec((BLK_C, 128), lambda i: (i, 0)),
            pl.BlockSpec((BLK_C, 128), lambda i: (i, 0)),
            pl.BlockSpec((1, 128), full),
            pl.BlockSpec((1, 128), full),
            pl.BlockSpec((1, 128), full),
            pl.BlockSpec((1, 128), full),
            pl.BlockSpec((128, HIDDEN), full),
            pl.BlockSpec((128, HIDDEN), full),
            pl.BlockSpec((1, HIDDEN), full),
            pl.BlockSpec((HIDDEN, HIDDEN), full),
            pl.BlockSpec((1, HIDDEN), full),
            pl.BlockSpec((HIDDEN, HIDDEN), full),
            pl.BlockSpec((1, HIDDEN), full),
            pl.BlockSpec((1, HIDDEN), full),
            pl.BlockSpec((1, 1), full),
        ],
        out_specs=pl.BlockSpec((BLK_C, 1), lambda i: (i, 0)),
        out_shape=jax.ShapeDtypeStruct((n, OUT_DIM), jnp.float32),
        compiler_params=pltpu.CompilerParams(
            dimension_semantics=("parallel",)),
    )(x, g0, g1, jnp.asarray(SCALE128), jnp.asarray(CX128),
      jnp.asarray(CY128), jnp.asarray(CZ128), M0, M1, b0, W1, b1, W2, b2,
      W3, b3)


NCHUNK = 4


def kernel(x, table, W0, b0, W1, b1, W2, b2, W3, b3):
    tab0 = table[:, 0]
    tab1 = table[:, 1]
    M0 = jnp.tile(W0[0::2], (8, 1))
    M1 = jnp.tile(W0[1::2], (8, 1))
    nc = N_POINTS // NCHUNK
    outs = []
    for i in range(NCHUNK):
        xs = jax.lax.slice(x, (i * nc, 0), ((i + 1) * nc, 3))
        idx = _stage_a(xs, nc)
        g0, g1 = _sc_gather(tab0, tab1, idx, nc)
        outs.append(_stage_c(
            xs, g0, g1, M0, M1, b0.reshape(1, HIDDEN), W1,
            b1.reshape(1, HIDDEN), W2, b2.reshape(1, HIDDEN),
            W3.reshape(1, HIDDEN), b3.reshape(1, 1), nc))
    return jnp.concatenate(outs, axis=0)
